# Initial kernel scaffold; baseline (speedup 1.0000x reference)
import functools

import jax
import jax.numpy as jnp
from jax import lax
from jax.experimental import pallas as pl
from jax.experimental.pallas import tpu as pltpu
from jax.experimental.pallas import tpu_sc as plsc

_VOC = 1_000_000
_DIM = 16
_N = 3_276_800

_NC = 2
_NS = 16
_NW = _NC * _NS
_PER_W = _N // _NW
_SEG = 128
_K = 8
_GROUP = _SEG * _K
_NBLK = 5
_BLK_ROWS = _PER_W // (_NBLK * _SEG)
_NGRP = _BLK_ROWS // _K


def _sc_body(x_ref, tbl_ref, out_ref, idx_v, rows_v, acc_v, sem):
    cid = lax.axis_index("c")
    sid = lax.axis_index("s")
    wid = sid * _NC + cid

    accs = (jnp.zeros((_DIM,), jnp.float32),) * _K
    for blk in range(_NBLK):
        pltpu.sync_copy(x_ref.at[wid, blk], idx_v)

        def group_body(g, accs):
            copies = [
                pltpu.async_copy(
                    tbl_ref.at[idx_v.at[g * _K + j]],
                    rows_v.at[pl.ds(j * _SEG, _SEG)],
                    sem,
                )
                for j in range(_K)
            ]
            for c in copies:
                c.wait()

            def acc_body(i, accs):
                return tuple(accs[j] + rows_v[i * _K + j] for j in range(_K))

            return lax.fori_loop(0, _GROUP // _K, acc_body, accs)

        accs = lax.fori_loop(0, _NGRP, group_body, accs)

    total = accs[0]
    for j in range(1, _K):
        total = total + accs[j]
    acc_v[...] = total * jnp.float32(1.0 / _N)
    pltpu.sync_copy(acc_v, out_ref.at[wid])


_sc_embedding_mean = functools.partial(
    pl.kernel,
    out_type=jax.ShapeDtypeStruct((_NW, _DIM), jnp.float32),
    mesh=plsc.VectorSubcoreMesh(
        core_axis_name="c",
        subcore_axis_name="s",
        num_cores=_NC,
        num_subcores=_NS,
    ),
    scratch_types=[
        pltpu.VMEM((_BLK_ROWS, _SEG), jnp.int32),
        pltpu.VMEM((_GROUP, _DIM), jnp.float32),
        pltpu.VMEM((_DIM,), jnp.float32),
        pltpu.SemaphoreType.DMA,
    ],
)(_sc_body)


def kernel(x, weight):
    x4 = x.astype(jnp.int32).reshape(_NW, _NBLK, _BLK_ROWS, _SEG)
    partials = _sc_embedding_mean(x4, weight)
    return jnp.sum(partials, axis=0, keepdims=True)


# speedup vs baseline: 2.8580x; 2.8580x over previous
import functools

import jax
import jax.numpy as jnp
from jax import lax
from jax.experimental import pallas as pl
from jax.experimental.pallas import tpu as pltpu
from jax.experimental.pallas import tpu_sc as plsc

_VOC = 1_000_000
_DIM = 16
_N = 3_276_800

_NC = 2
_NS = 16
_NW = _NC * _NS
_PER_W = _N // _NW
_SEG = 128
_K = 8
_GROUP = _SEG * _K
_NBLK = 5
_BLK_ROWS = _PER_W // (_NBLK * _SEG)
_NGRP = _BLK_ROWS // _K


def _sc_body(x_ref, tbl_ref, out_ref, idx_v, rows_v, acc_v, sem):
    cid = lax.axis_index("c")
    sid = lax.axis_index("s")
    wid = sid * _NC + cid

    accs = (jnp.zeros((_DIM,), jnp.float32),) * _K
    for blk in range(_NBLK):
        pltpu.sync_copy(x_ref.at[wid, blk], idx_v)

        def group_body(g, accs):
            copies = [
                pltpu.async_copy(
                    tbl_ref.at[idx_v.at[g * _K + j]],
                    rows_v.at[pl.ds(j * _SEG, _SEG)],
                    sem,
                )
                for j in range(_K)
            ]
            for c in copies:
                c.wait()

            def acc_body(i, accs):
                return tuple(accs[j] + rows_v[i * _K + j] for j in range(_K))

            return lax.fori_loop(0, _GROUP // _K, acc_body, accs)

        accs = lax.fori_loop(0, _NGRP, group_body, accs)

    total = accs[0]
    for j in range(1, _K):
        total = total + accs[j]
    acc_v[...] = total * jnp.float32(1.0 / _N)
    pltpu.sync_copy(acc_v, out_ref.at[wid])


_sc_embedding_mean = functools.partial(
    pl.kernel,
    out_type=jax.ShapeDtypeStruct((_NW, _DIM), jnp.float32),
    mesh=plsc.VectorSubcoreMesh(
        core_axis_name="c",
        subcore_axis_name="s",
        num_cores=_NC,
        num_subcores=_NS,
    ),
    scratch_types=[
        pltpu.VMEM((_BLK_ROWS, _SEG), jnp.int32),
        pltpu.VMEM((_GROUP, _DIM), jnp.float32),
        pltpu.VMEM((_DIM,), jnp.float32),
        pltpu.SemaphoreType.DMA,
    ],
    compiler_params=pltpu.CompilerParams(use_tc_tiling_on_sc=False),
)(_sc_body)


def kernel(x, weight):
    x4 = x.astype(jnp.int32).reshape(_NW, _NBLK, _BLK_ROWS, _SEG)
    partials = _sc_embedding_mean(x4, weight)
    return jnp.sum(partials, axis=0, keepdims=True)


# speedup vs baseline: 3.2141x; 1.1246x over previous
import functools

import jax
import jax.numpy as jnp
from jax import lax
from jax.experimental import pallas as pl
from jax.experimental.pallas import tpu as pltpu
from jax.experimental.pallas import tpu_sc as plsc

_VOC = 1_000_000
_DIM = 16
_N = 3_276_800

_NC = 2
_NS = 16
_NW = _NC * _NS
_PER_W = _N // _NW
_SEG = 128
_K = 4
_GROUP = _SEG * _K
_ACC = 8
_NBLK = 5
_BLK_IDX = _PER_W // _NBLK
_NGRP = _BLK_IDX // _GROUP


def _sc_body(x_ref, tbl_ref, out_ref, idx_v, rows_v, acc_v, sem0, sem1):
    cid = lax.axis_index("c")
    sid = lax.axis_index("s")
    wid = sid * _NC + cid
    base = wid * _PER_W

    sems = (sem0, sem1)

    def fire(g, b):
        for j in range(_K):
            pltpu.async_copy(
                tbl_ref.at[idx_v.at[pl.ds(g * _GROUP + j * _SEG, _SEG)]],
                rows_v.at[b, pl.ds(j * _SEG, _SEG)],
                sems[b],
            )

    def drain(b):
        pltpu.make_async_copy(
            tbl_ref.at[pl.ds(0, _GROUP)], rows_v.at[b], sems[b]
        ).wait()

    def accum(b, accs):
        def acc_body(i, accs):
            return tuple(
                accs[j] + rows_v[b, i * _ACC + j] for j in range(_ACC)
            )

        return lax.fori_loop(0, _GROUP // _ACC, acc_body, accs)

    accs = (jnp.zeros((_DIM,), jnp.float32),) * _ACC
    for blk in range(_NBLK):
        pltpu.sync_copy(x_ref.at[pl.ds(base + blk * _BLK_IDX, _BLK_IDX)], idx_v)
        fire(0, 0)

        def pair_body(i, accs):
            fire(2 * i + 1, 1)
            drain(0)
            accs = accum(0, accs)
            fire(2 * i + 2, 0)
            drain(1)
            accs = accum(1, accs)
            return accs

        accs = lax.fori_loop(0, _NGRP // 2 - 1, pair_body, accs)
        fire(_NGRP - 1, 1)
        drain(0)
        accs = accum(0, accs)
        drain(1)
        accs = accum(1, accs)

    total = accs[0]
    for j in range(1, _ACC):
        total = total + accs[j]
    acc_v[...] = total * jnp.float32(1.0 / _N)
    pltpu.sync_copy(acc_v, out_ref.at[wid])


_sc_embedding_mean = functools.partial(
    pl.kernel,
    out_type=jax.ShapeDtypeStruct((_NW, _DIM), jnp.float32),
    mesh=plsc.VectorSubcoreMesh(
        core_axis_name="c",
        subcore_axis_name="s",
        num_cores=_NC,
        num_subcores=_NS,
    ),
    scratch_types=[
        pltpu.VMEM((_BLK_IDX,), jnp.int32),
        pltpu.VMEM((2, _GROUP, _DIM), jnp.float32),
        pltpu.VMEM((_DIM,), jnp.float32),
        pltpu.SemaphoreType.DMA,
        pltpu.SemaphoreType.DMA,
    ],
    compiler_params=pltpu.CompilerParams(use_tc_tiling_on_sc=False),
)(_sc_body)


def kernel(x, weight):
    partials = _sc_embedding_mean(x.astype(jnp.int32), weight)
    return jnp.sum(partials, axis=0, keepdims=True)


# speedup vs baseline: 4.3741x; 1.3609x over previous
import functools

import jax
import jax.numpy as jnp
from jax import lax
from jax.experimental import pallas as pl
from jax.experimental.pallas import tpu as pltpu
from jax.experimental.pallas import tpu_sc as plsc

_VOC = 1_000_000
_DIM = 16
_N = 3_276_800

_NC = 2
_NS = 16
_NW = _NC * _NS
_VPAD = 1_048_576

_MESH = plsc.VectorSubcoreMesh(
    core_axis_name="c", subcore_axis_name="s", num_cores=_NC, num_subcores=_NS
)

_PER_W = _N // _NW
_SEG = 128
_HK = 8
_NBLK = 5
_BLK_ROWS = _PER_W // (_NBLK * _SEG)
_HGRP = _BLK_ROWS // _HK
_ZCH = 8192
_ZPT = _VPAD // _NS


def _hist_body(x_ref, cnt_ref, idx_v, ones_v, zero_v, cnt_sp, sem, zsem):
    cid = lax.axis_index("c")
    sid = lax.axis_index("s")
    wid = cid * _NS + sid

    def obody(i, _):
        ones_v[pl.ds(i * 16, 16)] = jnp.ones((16,), jnp.float32)
        return 0

    lax.fori_loop(0, _SEG // 16, obody, 0)

    def zbody(i, _):
        zero_v[pl.ds(i * 16, 16)] = jnp.zeros((16,), jnp.float32)
        return 0

    lax.fori_loop(0, _ZCH // 16, zbody, 0)

    zcopies = [
        pltpu.async_copy(
            zero_v, cnt_sp.at[pl.ds(sid * _ZPT + i * _ZCH, _ZCH)], zsem
        )
        for i in range(_ZPT // _ZCH)
    ]
    for c in zcopies:
        c.wait()
    plsc.subcore_barrier()

    def fire(g):
        for j in range(_HK):
            pltpu.async_copy(
                ones_v, cnt_sp.at[idx_v.at[g * _HK + j]], sem, add=True
            )

    def drain():
        pltpu.make_async_copy(
            cnt_ref.at[pl.ds(0, _HK * _SEG)],
            zero_v.at[pl.ds(0, _HK * _SEG)],
            sem,
        ).wait()

    for blk in range(_NBLK):
        pltpu.sync_copy(x_ref.at[wid, blk], idx_v)
        fire(0)

        def step(g, _):
            fire(g + 1)
            drain()
            return 0

        lax.fori_loop(0, _HGRP - 1, step, 0)
        drain()

    plsc.subcore_barrier()
    pltpu.sync_copy(
        cnt_sp.at[pl.ds(sid * _ZPT, _ZPT)],
        cnt_ref.at[pl.ds(cid * _VPAD + sid * _ZPT, _ZPT)],
    )


_sc_hist = functools.partial(
    pl.kernel,
    out_type=jax.ShapeDtypeStruct((_NC * _VPAD,), jnp.float32),
    mesh=_MESH,
    scratch_types=[
        pltpu.VMEM((_BLK_ROWS, _SEG), jnp.int32),
        pltpu.VMEM((_SEG,), jnp.float32),
        pltpu.VMEM((_ZCH,), jnp.float32),
        pltpu.VMEM_SHARED((_VPAD,), jnp.float32),
        pltpu.SemaphoreType.DMA,
        pltpu.SemaphoreType.DMA,
    ],
)(_hist_body)


_TB = 8000
_TBP = 8192
_TG = _VOC // _TB


def _tc_wsum_body(c_ref, w_ref, o_ref):
    i = pl.program_id(0)

    @pl.when(i == 0)
    def _init():
        o_ref[...] = jnp.zeros_like(o_ref)

    cb = c_ref[0][:, : _TB]
    wb = w_ref[...]
    o_ref[...] += jnp.dot(cb, wb, preferred_element_type=jnp.float32)


_tc_wsum = pl.pallas_call(
    _tc_wsum_body,
    grid=(_TG,),
    in_specs=[
        pl.BlockSpec((1, 1, _TBP), lambda i: (i, 0, 0)),
        pl.BlockSpec((_TB, _DIM), lambda i: (i, 0)),
    ],
    out_specs=pl.BlockSpec((1, _DIM), lambda i: (0, 0)),
    out_shape=jax.ShapeDtypeStruct((1, _DIM), jnp.float32),
)


def kernel(x, weight):
    x4 = x.astype(jnp.int32).reshape(_NW, _NBLK, _BLK_ROWS, _SEG)
    counts = _sc_hist(x4)
    c = counts[:_VOC] + counts[_VPAD:_VPAD + _VOC]
    c = jnp.pad(c.reshape(_TG, 1, _TB), ((0, 0), (0, 0), (0, _TBP - _TB)))
    return _tc_wsum(c, weight) * jnp.float32(1.0 / _N)


# speedup vs baseline: 18.4347x; 4.2145x over previous
import functools

import jax
import jax.numpy as jnp
from jax import lax
from jax.experimental import pallas as pl
from jax.experimental.pallas import tpu as pltpu
from jax.experimental.pallas import tpu_sc as plsc

_VOC = 1_000_000
_DIM = 16
_N = 3_276_800

_NC = 2
_NS = 16
_NW = _NC * _NS
_VPAD = 1_048_576

_MESH = plsc.VectorSubcoreMesh(
    core_axis_name="c", subcore_axis_name="s", num_cores=_NC, num_subcores=_NS
)

_PER_W = _N // _NW
_SEG = 128
_HK = 8
_NBLK = 5
_BLK_ROWS = _PER_W // (_NBLK * _SEG)
_HGRP = _BLK_ROWS // _HK
_ZCH = 8192
_ZPT = _VPAD // _NS


def _hist_body(x_ref, cnt_ref, idx_v, ones_v, zero_v, cnt_sp, sem, zsem):
    cid = lax.axis_index("c")
    sid = lax.axis_index("s")
    wid = cid * _NS + sid

    def obody(i, _):
        ones_v[pl.ds(i * 16, 16)] = jnp.ones((16,), jnp.float32)
        return 0

    lax.fori_loop(0, _SEG // 16, obody, 0)

    def zbody(i, _):
        zero_v[pl.ds(i * 16, 16)] = jnp.zeros((16,), jnp.float32)
        return 0

    lax.fori_loop(0, _ZCH // 16, zbody, 0)

    zcopies = [
        pltpu.async_copy(
            zero_v, cnt_sp.at[pl.ds(sid * _ZPT + i * _ZCH, _ZCH)], zsem
        )
        for i in range(_ZPT // _ZCH)
    ]
    for c in zcopies:
        c.wait()
    plsc.subcore_barrier()

    def fire(g):
        for j in range(_HK):
            pltpu.async_copy(
                ones_v, cnt_sp.at[idx_v.at[g * _HK + j]], sem, add=True
            )

    def drain():
        pltpu.make_async_copy(
            cnt_ref.at[pl.ds(0, _HK * _SEG)],
            zero_v.at[pl.ds(0, _HK * _SEG)],
            sem,
        ).wait()

    for blk in range(_NBLK):
        pltpu.sync_copy(x_ref.at[wid, blk], idx_v)
        fire(0)

        def step(g, _):
            fire(g + 1)
            drain()
            return 0

        lax.fori_loop(0, _HGRP - 1, step, 0)
        drain()

    plsc.subcore_barrier()
    pltpu.sync_copy(
        cnt_sp.at[pl.ds(sid * _ZPT, _ZPT)],
        cnt_ref.at[pl.ds(cid * _VPAD + sid * _ZPT, _ZPT)],
    )


_sc_hist = functools.partial(
    pl.kernel,
    out_type=jax.ShapeDtypeStruct((_NC * _VPAD,), jnp.float32),
    mesh=_MESH,
    scratch_types=[
        pltpu.VMEM((_BLK_ROWS, _SEG), jnp.int32),
        pltpu.VMEM((_SEG,), jnp.float32),
        pltpu.VMEM((_ZCH,), jnp.float32),
        pltpu.VMEM_SHARED((_VPAD,), jnp.float32),
        pltpu.SemaphoreType.DMA,
        pltpu.SemaphoreType.DMA,
    ],
)(_hist_body)


_TL = 65_536
_TG = _VPAD // _TL


def _tc_wsum_body(c_ref, w_ref, o_ref):
    i = pl.program_id(0)

    @pl.when(i == 0)
    def _init():
        o_ref[...] = jnp.zeros_like(o_ref)

    c = c_ref[0:1, :] + c_ref[1:2, :]
    wb = w_ref[...]
    v = i * _TL + jax.lax.broadcasted_iota(jnp.int32, (_DIM, _TL), 1)
    prod = jnp.where(v < _VOC, wb * c, jnp.float32(0.0))
    o_ref[...] += jnp.sum(prod, axis=1, keepdims=True)


_tc_wsum = pl.pallas_call(
    _tc_wsum_body,
    grid=(_TG,),
    in_specs=[
        pl.BlockSpec((_NC, _TL), lambda i: (0, i)),
        pl.BlockSpec((_DIM, _TL), lambda i: (0, i)),
    ],
    out_specs=pl.BlockSpec((_DIM, 1), lambda i: (0, 0)),
    out_shape=jax.ShapeDtypeStruct((_DIM, 1), jnp.float32),
)


def kernel(x, weight):
    x4 = x.astype(jnp.int32).reshape(_NW, _NBLK, _BLK_ROWS, _SEG)
    counts = _sc_hist(x4)
    o = _tc_wsum(counts.reshape(_NC, _VPAD), weight.T)
    return o.reshape(1, _DIM) * jnp.float32(1.0 / _N)


# speedup vs baseline: 22.3397x; 1.2118x over previous
import functools

import jax
import jax.numpy as jnp
from jax import lax
from jax.experimental import pallas as pl
from jax.experimental.pallas import tpu as pltpu
from jax.experimental.pallas import tpu_sc as plsc

_VOC = 1_000_000
_DIM = 16
_N = 3_276_800

_NC = 2
_NS = 16
_NW = _NC * _NS
_VPAD = 1_048_576

_MESH = plsc.VectorSubcoreMesh(
    core_axis_name="c", subcore_axis_name="s", num_cores=_NC, num_subcores=_NS
)

_PER_W = _N // _NW
_SEG = 128
_HK = 8
_NBLK = 5
_BLK_ROWS = _PER_W // (_NBLK * _SEG)
_HGRP = _BLK_ROWS // _HK
_ZCH = 8192
_ZPT = _VPAD // _NS


def _hist_body(x_ref, cnt_ref, idxa, idxb, ones_v, zero_v, cnt_sp, sem, zsem,
               ssem):
    cid = lax.axis_index("c")
    sid = lax.axis_index("s")
    wid = cid * _NS + sid

    def obody(i, _):
        ones_v[pl.ds(i * 16, 16)] = jnp.ones((16,), jnp.float32)
        return 0

    lax.fori_loop(0, _SEG // 16, obody, 0)

    def zbody(i, _):
        zero_v[pl.ds(i * 16, 16)] = jnp.zeros((16,), jnp.float32)
        return 0

    lax.fori_loop(0, _ZCH // 16, zbody, 0)

    stage = pltpu.async_copy(x_ref.at[wid, 0], idxa, ssem)
    zcopies = [
        pltpu.async_copy(
            zero_v, cnt_sp.at[pl.ds(sid * _ZPT + i * _ZCH, _ZCH)], zsem
        )
        for i in range(_ZPT // _ZCH)
    ]
    for c in zcopies:
        c.wait()
    plsc.subcore_barrier()

    def fire(idx_v, g):
        for j in range(_HK):
            pltpu.async_copy(
                ones_v, cnt_sp.at[idx_v.at[g * _HK + j]], sem, add=True
            )

    def drain():
        pltpu.make_async_copy(
            cnt_ref.at[0, pl.ds(0, _HK * _SEG)],
            zero_v.at[pl.ds(0, _HK * _SEG)],
            sem,
        ).wait()

    bufs = (idxa, idxb)
    for blk in range(_NBLK):
        idx_v = bufs[blk % 2]
        stage.wait()
        if blk + 1 < _NBLK:
            stage = pltpu.async_copy(
                x_ref.at[wid, blk + 1], bufs[(blk + 1) % 2], ssem
            )
        fire(idx_v, 0)
        fire(idx_v, 1)

        def step(g, _):
            fire(idx_v, g + 2)
            drain()
            return 0

        lax.fori_loop(0, _HGRP - 2, step, 0)
        drain()
        drain()

    plsc.subcore_barrier()
    pltpu.sync_copy(
        cnt_sp.at[pl.ds(sid * _ZPT, _ZPT)],
        cnt_ref.at[cid, pl.ds(sid * _ZPT, _ZPT)],
    )


_sc_hist = functools.partial(
    pl.kernel,
    out_type=jax.ShapeDtypeStruct((_NC, _VPAD), jnp.float32),
    mesh=_MESH,
    scratch_types=[
        pltpu.VMEM((_BLK_ROWS, _SEG), jnp.int32),
        pltpu.VMEM((_BLK_ROWS, _SEG), jnp.int32),
        pltpu.VMEM((_SEG,), jnp.float32),
        pltpu.VMEM((_ZCH,), jnp.float32),
        pltpu.VMEM_SHARED((_VPAD,), jnp.float32),
        pltpu.SemaphoreType.DMA,
        pltpu.SemaphoreType.DMA,
        pltpu.SemaphoreType.DMA,
    ],
)(_hist_body)


_TL = 65_536
_TG = _VPAD // _TL


def _tc_wsum_body(c_ref, w_ref, o_ref):
    i = pl.program_id(0)

    @pl.when(i == 0)
    def _init():
        o_ref[...] = jnp.zeros_like(o_ref)

    c = c_ref[0:1, :] + c_ref[1:2, :]
    wb = w_ref[...]
    v = i * _TL + jax.lax.broadcasted_iota(jnp.int32, (_DIM, _TL), 1)
    prod = jnp.where(v < _VOC, wb * c, jnp.float32(0.0))
    o_ref[...] += jnp.sum(prod, axis=1, keepdims=True)


_tc_wsum = pl.pallas_call(
    _tc_wsum_body,
    grid=(_TG,),
    in_specs=[
        pl.BlockSpec((_NC, _TL), lambda i: (0, i)),
        pl.BlockSpec((_DIM, _TL), lambda i: (0, i)),
    ],
    out_specs=pl.BlockSpec((_DIM, 1), lambda i: (0, 0)),
    out_shape=jax.ShapeDtypeStruct((_DIM, 1), jnp.float32),
)


def kernel(x, weight):
    x4 = x.astype(jnp.int32).reshape(_NW, _NBLK, _BLK_ROWS, _SEG)
    counts = _sc_hist(x4)
    o = _tc_wsum(counts, weight.T)
    return o.reshape(1, _DIM) * jnp.float32(1.0 / _N)
